# Initial kernel scaffold; baseline (speedup 1.0000x reference)
import jax
import jax.numpy as jnp
from jax.experimental import pallas as pl

H, W = 24, 80
HW = H * W
PAD = 96
XROWS = 2 * PAD + HW
C_IN = 1024
C_OUT = 256
G = 32
CPG = C_OUT // G
TH, TW = 4, 4
NTY, NTX = H // TH, W // TW
NTOK = NTY * NTX
NCLUST = 100
ALPHA = 1.0

_HI = jax.lax.Precision.HIGHEST
_f32 = jnp.float32


def _dense_body(xpad_ref, w1_ref, b1_ref, gnw_ref, gnb_ref, w2_ref, b2_ref,
                cal_ref, comb_ref, tok_ref, fc_ref):
    colmod = jax.lax.broadcasted_iota(jnp.int32, (HW, 1), 0) % W

    acc = jnp.zeros((HW, C_OUT), _f32)
    for dx in range(3):
        acc_dx = jnp.zeros((HW, C_OUT), _f32)
        for dy in range(3):
            t = dy * 3 + dx
            start = PAD + (dy - 1) * W + (dx - 1)
            x_t = xpad_ref[0, pl.ds(start, HW), :]
            acc_dx = acc_dx + jnp.dot(
                x_t, w1_ref[pl.ds(t * C_IN, C_IN), :],
                preferred_element_type=_f32, precision=_HI)
        if dx == 0:
            acc_dx = jnp.where(colmod != 0, acc_dx, 0.0)
        elif dx == 2:
            acc_dx = jnp.where(colmod != W - 1, acc_dx, 0.0)
        acc = acc + acc_dx
    h = acc + b1_ref[...]

    n = _f32(HW * CPG)
    s1 = jnp.sum(h, axis=0, keepdims=True)
    s2 = jnp.sum(h * h, axis=0, keepdims=True)
    gi = jax.lax.broadcasted_iota(jnp.int32, (C_OUT, C_OUT), 0) // CPG
    gj = jax.lax.broadcasted_iota(jnp.int32, (C_OUT, C_OUT), 1) // CPG
    gmat = (gi == gj).astype(_f32)
    g1 = jnp.dot(s1, gmat, preferred_element_type=_f32, precision=_HI)
    g2 = jnp.dot(s2, gmat, preferred_element_type=_f32, precision=_HI)
    mu = g1 / n
    var = g2 / n - mu * mu
    inv = jax.lax.rsqrt(var + 1e-5)
    hn = (h - mu) * (inv * gnw_ref[...]) + gnb_ref[...]
    hr = jnp.maximum(hn, 0.0)

    logit = jnp.sum(hr * w2_ref[...], axis=1, keepdims=True) + b2_ref[0, 0]
    heat = jax.nn.sigmoid(logit)

    fy = cal_ref[0, 0, 0]
    cy = _f32(H) * cal_ref[0, 0, 1] / 375.0
    v = (jax.lax.broadcasted_iota(jnp.int32, (HW, 1), 0) // W).astype(_f32)
    depth = -jax.nn.relu(500.0 * (v - cy) / (fy * _f32(H)))
    cf = depth + ALPHA * heat

    p_i = jax.lax.broadcasted_iota(jnp.int32, (HW, W), 0)
    x_j = jax.lax.broadcasted_iota(jnp.int32, (HW, W), 1)
    bmat = jnp.where(p_i % W == x_j, 1.0, 0.0) * cf
    y_i = jax.lax.broadcasted_iota(jnp.int32, (H, HW), 0)
    p_j = jax.lax.broadcasted_iota(jnp.int32, (H, HW), 1)
    rsel = (p_j // W == y_i).astype(_f32)
    comb_ref[0] = jnp.dot(rsel, bmat, preferred_element_type=_f32,
                          precision=_HI)

    tt_i = jax.lax.broadcasted_iota(jnp.int32, (NTOK, HW), 0)
    tp_j = jax.lax.broadcasted_iota(jnp.int32, (NTOK, HW), 1)
    t_of_p = (tp_j // (TH * W)) * NTX + (tp_j % W) // TW
    pm = jnp.where(t_of_p == tt_i, 1.0 / (TH * TW), 0.0)
    x_c = xpad_ref[0, pl.ds(PAD, HW), :]
    tokens = jnp.dot(pm, x_c, preferred_element_type=_f32, precision=_HI)
    tok_ref[0] = tokens

    ts_col = jnp.dot(pm, cf, preferred_element_type=_f32, precision=_HI)
    i_t = jax.lax.broadcasted_iota(jnp.int32, (NTOK, NTOK), 0)
    j_t = jax.lax.broadcasted_iota(jnp.int32, (NTOK, NTOK), 1)
    eye = (i_t == j_t).astype(_f32)
    ts_row = jnp.dot(jnp.ones((1, NTOK), _f32), eye * ts_col,
                     preferred_element_type=_f32, precision=_HI)
    beats = ((ts_col > ts_row) | ((ts_col == ts_row) & (i_t < j_t)))
    rank = jnp.sum(beats.astype(_f32), axis=0, keepdims=True)
    k_i = jax.lax.broadcasted_iota(jnp.int32, (NCLUST, NTOK), 0).astype(_f32)
    sel = (k_i == rank).astype(_f32)
    fc_ref[0] = jnp.dot(sel, tokens, preferred_element_type=_f32,
                        precision=_HI)


@jax.jit
def kernel(features, calibs, W1, b1, gn_w, gn_b, W2, b2):
    BS = features.shape[0]
    xf = features.transpose(0, 2, 3, 1).reshape(BS, HW, C_IN)
    xpad = jnp.pad(xf, ((0, 0), (PAD, PAD), (0, 0)))
    w1r = W1.transpose(2, 3, 1, 0).reshape(9 * C_IN, C_OUT)
    cal2 = jnp.stack([calibs[:, 1, 1], calibs[:, 1, 2]],
                     axis=-1).reshape(BS, 1, 2)

    comb, tokens, fc = pl.pallas_call(
        _dense_body,
        grid=(BS,),
        in_specs=[
            pl.BlockSpec((1, XROWS, C_IN), lambda b: (b, 0, 0)),
            pl.BlockSpec((9 * C_IN, C_OUT), lambda b: (0, 0)),
            pl.BlockSpec((1, C_OUT), lambda b: (0, 0)),
            pl.BlockSpec((1, C_OUT), lambda b: (0, 0)),
            pl.BlockSpec((1, C_OUT), lambda b: (0, 0)),
            pl.BlockSpec((1, C_OUT), lambda b: (0, 0)),
            pl.BlockSpec((1, 1), lambda b: (0, 0)),
            pl.BlockSpec((1, 1, 2), lambda b: (b, 0, 0)),
        ],
        out_specs=[
            pl.BlockSpec((1, H, W), lambda b: (b, 0, 0)),
            pl.BlockSpec((1, NTOK, C_IN), lambda b: (b, 0, 0)),
            pl.BlockSpec((1, NCLUST, C_IN), lambda b: (b, 0, 0)),
        ],
        out_shape=[
            jax.ShapeDtypeStruct((BS, H, W), _f32),
            jax.ShapeDtypeStruct((BS, NTOK, C_IN), _f32),
            jax.ShapeDtypeStruct((BS, NCLUST, C_IN), _f32),
        ],
    )(xpad, w1r, b1.reshape(1, C_OUT), gn_w.reshape(1, C_OUT),
      gn_b.reshape(1, C_OUT), W2.reshape(1, C_OUT), b2.reshape(1, 1), cal2)

    ii = jnp.arange(NTY) * TH + TH // 2
    jj = jnp.arange(NTX) * TW + TW // 2
    pos = jnp.stack(jnp.meshgrid(ii, jj, indexing='ij'),
                    axis=-1).reshape(NTOK, 2).astype(jnp.int32)
    token_positions = jnp.broadcast_to(pos[None], (BS, NTOK, 2))
    return comb, fc, tokens, token_positions


# speedup vs baseline: 1.0360x; 1.0360x over previous
import jax
import jax.numpy as jnp
from jax.experimental import pallas as pl

H, W = 24, 80
HW = H * W
PAD = 96
XROWS = 2 * PAD + HW
C_IN = 1024
C_OUT = 256
TH, TW = 4, 4
NTY, NTX = H // TH, W // TW
NTOK = NTY * NTX
NCLUST = 100
ALPHA = 1.0
CHUNK = 240
NCH = HW // CHUNK

_HI = jax.lax.Precision.HIGHEST
_f32 = jnp.float32


def _conv_body(xpad_ref, w9_ref, h_ref, tok_ref):
    tok_acc = jnp.zeros((NTOK, C_IN), _f32)
    for c in range(NCH):
        base = c * CHUNK
        rm = jax.lax.broadcasted_iota(jnp.int32, (CHUNK, 1), 0) + base
        colmod = rm % W
        pieces = []
        for t in range(9):
            dy, dx = t // 3, t % 3
            start = PAD + (dy - 1) * W + (dx - 1) + base
            x_t = xpad_ref[0, pl.ds(start, CHUNK), :]
            if dx == 0:
                x_t = x_t * (colmod != 0).astype(_f32)
            elif dx == 2:
                x_t = x_t * (colmod != W - 1).astype(_f32)
            pieces.append(x_t)
        x9 = jnp.concatenate(pieces, axis=1)
        h_ref[0, pl.ds(base, CHUNK), :] = jnp.dot(
            x9, w9_ref[...], preferred_element_type=_f32,
            precision=jax.lax.Precision.DEFAULT)

        x_c = xpad_ref[0, pl.ds(PAD + base, CHUNK), :]
        tt_i = jax.lax.broadcasted_iota(jnp.int32, (NTOK, CHUNK), 0)
        tp_j = jax.lax.broadcasted_iota(jnp.int32, (NTOK, CHUNK), 1) + base
        t_of_p = (tp_j // (TH * W)) * NTX + (tp_j % W) // TW
        pm = jnp.where(t_of_p == tt_i, 1.0 / (TH * TW), 0.0)
        tok_acc = tok_acc + jnp.dot(pm, x_c, preferred_element_type=_f32,
                                    precision=_HI)
    tok_ref[0] = tok_acc


def _select_body(ts_ref, tok_ref, fc_ref):
    ts_row = ts_ref[0]
    i_t = jax.lax.broadcasted_iota(jnp.int32, (NTOK, NTOK), 0)
    j_t = jax.lax.broadcasted_iota(jnp.int32, (NTOK, NTOK), 1)
    eye = (i_t == j_t).astype(_f32)
    ts_col = jnp.dot(eye * ts_row, jnp.ones((NTOK, 1), _f32),
                     preferred_element_type=_f32, precision=_HI)
    beats = ((ts_col > ts_row) | ((ts_col == ts_row) & (i_t < j_t)))
    rank = jnp.sum(beats.astype(_f32), axis=0, keepdims=True)
    k_i = jax.lax.broadcasted_iota(jnp.int32, (NCLUST, NTOK), 0).astype(_f32)
    sel = (k_i == rank).astype(_f32)
    fc_ref[0] = jnp.dot(sel, tok_ref[0], preferred_element_type=_f32,
                        precision=_HI)


@jax.jit
def kernel(features, calibs, W1, b1, gn_w, gn_b, W2, b2):
    BS = features.shape[0]
    xf = features.transpose(0, 2, 3, 1).reshape(BS, HW, C_IN)
    xpad = jnp.pad(xf, ((0, 0), (PAD, PAD), (0, 0)))
    w9 = W1.transpose(2, 3, 1, 0).reshape(9 * C_IN, C_OUT)

    h, tokens = pl.pallas_call(
        _conv_body,
        grid=(BS,),
        in_specs=[
            pl.BlockSpec((1, XROWS, C_IN), lambda b: (b, 0, 0)),
            pl.BlockSpec((9 * C_IN, C_OUT), lambda b: (0, 0)),
        ],
        out_specs=[
            pl.BlockSpec((1, HW, C_OUT), lambda b: (b, 0, 0)),
            pl.BlockSpec((1, NTOK, C_IN), lambda b: (b, 0, 0)),
        ],
        out_shape=[
            jax.ShapeDtypeStruct((BS, HW, C_OUT), _f32),
            jax.ShapeDtypeStruct((BS, NTOK, C_IN), _f32),
        ],
    )(xpad, w9)

    h4 = h.reshape(BS, H, W, C_OUT).transpose(0, 3, 1, 2)
    h4 = h4 + b1.reshape(1, -1, 1, 1)
    Gn = 32
    hg = h4.reshape(BS, Gn, C_OUT // Gn, H, W)
    mu = hg.mean(axis=(2, 3, 4), keepdims=True)
    var = hg.var(axis=(2, 3, 4), keepdims=True)
    hg = (hg - mu) / jnp.sqrt(var + 1e-5)
    h4 = hg.reshape(BS, C_OUT, H, W) * gn_w.reshape(1, -1, 1, 1) \
        + gn_b.reshape(1, -1, 1, 1)
    h4 = jax.nn.relu(h4)
    h4 = jax.lax.conv_general_dilated(
        h4, W2, (1, 1), ((0, 0), (0, 0)),
        dimension_numbers=('NCHW', 'OIHW', 'NCHW'))
    h4 = h4 + b2.reshape(1, -1, 1, 1)
    heatmap = jax.nn.sigmoid(h4)[:, 0]

    v = jnp.arange(H, dtype=_f32).reshape(1, H, 1)
    v = jnp.broadcast_to(v, (BS, H, W))
    fy = calibs[:, 1, 1].reshape(-1, 1, 1)
    cy = calibs[:, 1, 2].reshape(-1, 1, 1)
    cy = H * cy / 375.0
    depth_scores = -jax.nn.relu(500.0 * (v - cy) / (fy * H))
    combined = depth_scores + ALPHA * heatmap

    token_scores = combined.reshape(BS, NTY, TH, NTX, TW).mean(
        axis=(2, 4)).reshape(BS, NTY * NTX)

    fc = pl.pallas_call(
        _select_body,
        grid=(BS,),
        in_specs=[
            pl.BlockSpec((1, 1, NTOK), lambda b: (b, 0, 0)),
            pl.BlockSpec((1, NTOK, C_IN), lambda b: (b, 0, 0)),
        ],
        out_specs=pl.BlockSpec((1, NCLUST, C_IN), lambda b: (b, 0, 0)),
        out_shape=jax.ShapeDtypeStruct((BS, NCLUST, C_IN), _f32),
    )(token_scores.reshape(BS, 1, NTOK), tokens)

    ii = jnp.arange(NTY) * TH + TH // 2
    jj = jnp.arange(NTX) * TW + TW // 2
    pos = jnp.stack(jnp.meshgrid(ii, jj, indexing='ij'),
                    axis=-1).reshape(NTOK, 2).astype(jnp.int32)
    token_positions = jnp.broadcast_to(pos[None], (BS, NTOK, 2))
    return combined, fc, tokens, token_positions
